# baseline (speedup 1.0000x reference)
import functools
import jax
import jax.numpy as jnp
from jax import lax
from jax.experimental import pallas as pl
from jax.experimental.pallas import tpu as pltpu
from jax.experimental.pallas import tpu_sc as plsc

N_AUTH = 1_000_000
DIM = 16
N_PLANES = 4
NROW = N_PLANES * DIM
BATCH = 16384
NC, NS, L = 2, 16, 16
NW = NC * NS
BPW = BATCH // NW
NG = BPW // L
BLK = 128
NBLK_FULL = N_AUTH // BLK
BPT = NBLK_FULL // NW
NEXTRA = NBLK_FULL - BPT * NW
TAIL = N_AUTH - NBLK_FULL * BLK
ETROWS = (NBLK_FULL + 1) * DIM
GROWS = ETROWS * BLK // L


TCB = 1024
TCGRID = (N_AUTH + TCB - 1) // TCB


def _tc_combine_body(wr, xr, outr):
    ew = jnp.exp(wr[...])
    wn = ew / jnp.sum(ew, axis=1, keepdims=True)
    wflat = wn.reshape(4, 1, 1)
    x4 = xr[...].reshape(N_PLANES, DIM, TCB)
    e = jnp.sum(x4 * wflat, axis=0)
    for bb in range(TCB // BLK):
        outr[bb * DIM:(bb + 1) * DIM, :] = e[:, bb * BLK:(bb + 1) * BLK]


def _gather_body(gt, idx_i, idx_j, params, out,
                 idxr, idxl, rgb, pv, mat, outv, sa, sb):
    sems = (sa, sb)
    wid = lax.axis_index("s") * NC + lax.axis_index("c")
    base = wid * BPW
    iota = lax.iota(jnp.int32, L)

    pltpu.sync_copy(params, pv)
    itcv = plsc.load_gather(pv, [jnp.full((L,), 2, jnp.int32),
                                 jnp.full((L,), 0, jnp.int32)])
    slpv = plsc.load_gather(pv, [jnp.full((L,), 2, jnp.int32),
                                 jnp.full((L,), 1, jnp.int32)])

    pltpu.sync_copy(idx_i.at[pl.ds(base, BPW)], idxr.at[0])
    pltpu.sync_copy(idx_j.at[pl.ds(base, BPW)], idxr.at[1])

    def fire(g, b, sem):
        for s in range(2):
            nv = idxr[s, pl.ds(g * L, L)]
            bv = (nv // BLK) * BLK + (nv % BLK) // L
            for u in range(L):
                gr = iota * 8 + jnp.full((L,), bv[u], jnp.int32)
                idxl[b, s, u // 8, pl.ds((u % 8) * L, L)] = gr
        for s in range(2):
            for c in range(2):
                pltpu.async_copy(gt.at[idxl.at[b, s, c]],
                                 rgb.at[b, s, c], sem)

    def drain(b, sem):
        for s in range(2):
            for c in range(2):
                pltpu.make_async_copy(gt.at[pl.ds(0, 8 * L)],
                                      rgb.at[b, s, c], sem).wait()

    def compute(g, b):
        nvi = idxr[0, pl.ds(g * L, L)]
        nvj = idxr[1, pl.ds(g * L, L)]
        li = nvi % L
        lj = nvj % L
        bb = jnp.full((L,), b, jnp.int32)
        s0 = jnp.full((L,), 0, jnp.int32)
        s1 = jnp.full((L,), 1, jnp.int32)
        for u in range(L):
            cc = jnp.full((L,), u // 8, jnp.int32)
            rows = iota + (u % 8) * L
            zi = plsc.load_gather(
                rgb, [bb, s0, cc, rows, jnp.full((L,), li[u], jnp.int32)])
            zj = plsc.load_gather(
                rgb, [bb, s1, cc, rows, jnp.full((L,), lj[u], jnp.int32)])
            dd = zi - zj
            mat[u, :] = dd * dd
        dv = plsc.load_gather(mat, [iota, jnp.full((L,), 0, jnp.int32)])
        for d in range(1, DIM):
            dv = dv + plsc.load_gather(
                mat, [iota, jnp.full((L,), d, jnp.int32)])
        z = itcv - slpv * dv * (1.0 / DIM)
        outv[pl.ds(g * L, L)] = 1.0 / (1.0 + jnp.exp(-z))

    fire(0, 0, sems[0])

    def pair(gp, carry):
        g0 = gp * 2
        fire(g0 + 1, 1, sems[1])
        drain(0, sems[0])
        compute(g0, 0)
        fire(g0 + 2, 0, sems[0])
        drain(1, sems[1])
        compute(g0 + 1, 1)
        return carry

    lax.fori_loop(0, NG // 2 - 1, pair, 0)
    fire(NG - 1, 1, sems[1])
    drain(0, sems[0])
    compute(NG - 2, 0)
    drain(1, sems[1])
    compute(NG - 1, 1)

    pltpu.sync_copy(outv, out.at[pl.ds(base, BPW)])


@jax.jit
def _run(ft, idx_i, idx_j, fwraw, params):
    mesh = plsc.VectorSubcoreMesh(core_axis_name="c", subcore_axis_name="s")
    etab = pl.pallas_call(
        _tc_combine_body,
        grid=(TCGRID,),
        in_specs=[
            pl.BlockSpec((2, 2), lambda g: (0, 0)),
            pl.BlockSpec((NROW, TCB), lambda g: (0, g)),
        ],
        out_specs=pl.BlockSpec((BLK, BLK), lambda g: (g, 0)),
        out_shape=jax.ShapeDtypeStruct((ETROWS, BLK), jnp.float32),
    )(fwraw, ft)

    gt = etab.reshape(GROWS, DIM)
    gather = functools.partial(
        pl.kernel,
        mesh=mesh,
        out_type=jax.ShapeDtypeStruct((BATCH,), jnp.float32),
        scratch_types=[
            pltpu.VMEM((2, BPW), jnp.int32),
            pltpu.VMEM((2, 2, 2, 8 * L), jnp.int32),
            pltpu.VMEM((2, 2, 2, 8 * L, L), jnp.float32),
            pltpu.VMEM((3, L), jnp.float32),
            pltpu.VMEM((L, L), jnp.float32),
            pltpu.VMEM((BPW,), jnp.float32),
            pltpu.SemaphoreType.DMA,
            pltpu.SemaphoreType.DMA,
        ],
        compiler_params=pltpu.CompilerParams(
            needs_layout_passes=False, use_tc_tiling_on_sc=False),
    )(_gather_body)
    return gather(gt, idx_i, idx_j, params)


def kernel(features, feature_weights, intercept, slope, idx_i, idx_j):
    ft = jnp.transpose(features, (0, 1, 3, 2)).reshape(NROW, N_AUTH)
    fw = feature_weights.reshape(-1).astype(jnp.float32)
    pad = jnp.zeros((L - 4,), jnp.float32)
    wa = jnp.concatenate([fw, pad])
    wb = jnp.concatenate([fw[1::2].reshape(2, 1),
                          fw[0::2].reshape(2, 1)], axis=1).reshape(-1)
    wb = jnp.concatenate([wb, pad])
    sc = jnp.concatenate([jnp.float32(intercept).reshape(1),
                          jnp.float32(slope).reshape(1),
                          jnp.zeros((L - 2,), jnp.float32)])
    params = jnp.stack([wa, wb, sc])
    return _run(ft, idx_i.astype(jnp.int32), idx_j.astype(jnp.int32),
                feature_weights.astype(jnp.float32), params)


# speedup vs baseline: 3.3521x; 3.3521x over previous
import functools
import jax
import jax.numpy as jnp
from jax import lax
from jax.experimental import pallas as pl
from jax.experimental.pallas import tpu as pltpu
from jax.experimental.pallas import tpu_sc as plsc

N_AUTH = 1_000_000
DIM = 16
N_PLANES = 4
NROW = N_PLANES * DIM
BATCH = 16384
NC, NS, L = 2, 16, 16
NW = NC * NS
BPW = BATCH // NW
NG = BPW // L
BLK = 128
NBLK_FULL = N_AUTH // BLK
BPT = NBLK_FULL // NW
NEXTRA = NBLK_FULL - BPT * NW
TAIL = N_AUTH - NBLK_FULL * BLK
ETROWS = (NBLK_FULL + 1) * DIM
GROWS = ETROWS * BLK // L


TCB = 8192
TCGRID = (N_AUTH + TCB - 1) // TCB


def _tc_combine_body(wr, xr, outr):
    ew = jnp.exp(wr[...])
    wn = ew / jnp.sum(ew, axis=1, keepdims=True)
    wflat = wn.reshape(4, 1, 1)
    x4 = xr[...].reshape(N_PLANES, DIM, TCB)
    e = jnp.sum(x4 * wflat, axis=0)
    for bb in range(TCB // BLK):
        outr[bb * DIM:(bb + 1) * DIM, :] = e[:, bb * BLK:(bb + 1) * BLK]


def _gather_body(gt, idx_i, idx_j, params, out,
                 idxr, idxl, rgb, pv, mat, outv, sa, sb):
    sems = (sa, sb)
    wid = lax.axis_index("s") * NC + lax.axis_index("c")
    base = wid * BPW
    iota = lax.iota(jnp.int32, L)

    pltpu.sync_copy(params, pv)
    itcv = plsc.load_gather(pv, [jnp.full((L,), 2, jnp.int32),
                                 jnp.full((L,), 0, jnp.int32)])
    slpv = plsc.load_gather(pv, [jnp.full((L,), 2, jnp.int32),
                                 jnp.full((L,), 1, jnp.int32)])

    pltpu.sync_copy(idx_i.at[pl.ds(base, BPW)], idxr.at[0])
    pltpu.sync_copy(idx_j.at[pl.ds(base, BPW)], idxr.at[1])

    def fire(g, b, sem):
        for s in range(2):
            nv = idxr[s, pl.ds(g * L, L)]
            bv = (nv // BLK) * BLK + (nv % BLK) // L
            for u in range(L):
                gr = iota * 8 + jnp.full((L,), bv[u], jnp.int32)
                idxl[b, s, u // 8, pl.ds((u % 8) * L, L)] = gr
        for s in range(2):
            for c in range(2):
                pltpu.async_copy(gt.at[idxl.at[b, s, c]],
                                 rgb.at[b, s, c], sem)

    def drain(b, sem):
        for s in range(2):
            for c in range(2):
                pltpu.make_async_copy(gt.at[pl.ds(0, 8 * L)],
                                      rgb.at[b, s, c], sem).wait()

    def compute(g, b):
        nvi = idxr[0, pl.ds(g * L, L)]
        nvj = idxr[1, pl.ds(g * L, L)]
        li = nvi % L
        lj = nvj % L
        bb = jnp.full((L,), b, jnp.int32)
        s0 = jnp.full((L,), 0, jnp.int32)
        s1 = jnp.full((L,), 1, jnp.int32)
        for u in range(L):
            cc = jnp.full((L,), u // 8, jnp.int32)
            rows = iota + (u % 8) * L
            zi = plsc.load_gather(
                rgb, [bb, s0, cc, rows, jnp.full((L,), li[u], jnp.int32)])
            zj = plsc.load_gather(
                rgb, [bb, s1, cc, rows, jnp.full((L,), lj[u], jnp.int32)])
            dd = zi - zj
            mat[u, :] = dd * dd
        dv = plsc.load_gather(mat, [iota, jnp.full((L,), 0, jnp.int32)])
        for d in range(1, DIM):
            dv = dv + plsc.load_gather(
                mat, [iota, jnp.full((L,), d, jnp.int32)])
        z = itcv - slpv * dv * (1.0 / DIM)
        outv[pl.ds(g * L, L)] = 1.0 / (1.0 + jnp.exp(-z))

    fire(0, 0, sems[0])

    def pair(gp, carry):
        g0 = gp * 2
        fire(g0 + 1, 1, sems[1])
        drain(0, sems[0])
        compute(g0, 0)
        fire(g0 + 2, 0, sems[0])
        drain(1, sems[1])
        compute(g0 + 1, 1)
        return carry

    lax.fori_loop(0, NG // 2 - 1, pair, 0)
    fire(NG - 1, 1, sems[1])
    drain(0, sems[0])
    compute(NG - 2, 0)
    drain(1, sems[1])
    compute(NG - 1, 1)

    pltpu.sync_copy(outv, out.at[pl.ds(base, BPW)])


@jax.jit
def _run(ft, idx_i, idx_j, fwraw, params):
    mesh = plsc.VectorSubcoreMesh(core_axis_name="c", subcore_axis_name="s")
    etab = pl.pallas_call(
        _tc_combine_body,
        grid=(TCGRID,),
        in_specs=[
            pl.BlockSpec((2, 2), lambda g: (0, 0)),
            pl.BlockSpec((NROW, TCB), lambda g: (0, g)),
        ],
        out_specs=pl.BlockSpec((TCB * DIM // BLK, BLK), lambda g: (g, 0)),
        out_shape=jax.ShapeDtypeStruct((ETROWS, BLK), jnp.float32),
    )(fwraw, ft)

    gt = etab.reshape(GROWS, DIM)
    gather = functools.partial(
        pl.kernel,
        mesh=mesh,
        out_type=jax.ShapeDtypeStruct((BATCH,), jnp.float32),
        scratch_types=[
            pltpu.VMEM((2, BPW), jnp.int32),
            pltpu.VMEM((2, 2, 2, 8 * L), jnp.int32),
            pltpu.VMEM((2, 2, 2, 8 * L, L), jnp.float32),
            pltpu.VMEM((3, L), jnp.float32),
            pltpu.VMEM((L, L), jnp.float32),
            pltpu.VMEM((BPW,), jnp.float32),
            pltpu.SemaphoreType.DMA,
            pltpu.SemaphoreType.DMA,
        ],
        compiler_params=pltpu.CompilerParams(
            needs_layout_passes=False, use_tc_tiling_on_sc=False),
    )(_gather_body)
    return gather(gt, idx_i, idx_j, params)


def kernel(features, feature_weights, intercept, slope, idx_i, idx_j):
    ft = jnp.transpose(features, (0, 1, 3, 2)).reshape(NROW, N_AUTH)
    fw = feature_weights.reshape(-1).astype(jnp.float32)
    pad = jnp.zeros((L - 4,), jnp.float32)
    wa = jnp.concatenate([fw, pad])
    wb = jnp.concatenate([fw[1::2].reshape(2, 1),
                          fw[0::2].reshape(2, 1)], axis=1).reshape(-1)
    wb = jnp.concatenate([wb, pad])
    sc = jnp.concatenate([jnp.float32(intercept).reshape(1),
                          jnp.float32(slope).reshape(1),
                          jnp.zeros((L - 2,), jnp.float32)])
    params = jnp.stack([wa, wb, sc])
    return _run(ft, idx_i.astype(jnp.int32), idx_j.astype(jnp.int32),
                feature_weights.astype(jnp.float32), params)
